# baseline (speedup 1.0000x reference)
import functools

import jax
import jax.numpy as jnp
from jax import lax
from jax.experimental import pallas as pl
from jax.experimental.pallas import tpu as pltpu
from jax.experimental.pallas import tpu_sc as plsc

_OBS_DIM = 512
_HIDDEN_DIM = 512
_NUM_ENVS = 64
_N_AGENTS = 128
_T2 = 0.04000000283122063
_ZROW = _NUM_ENVS * _N_AGENTS
_NW = 32
_EPW = _NUM_ENVS // _NW
_F = _OBS_DIM
_FC = _F // 16


def _bcast_lane(v, lane_vec):
    return _gather_idx(v, lane_vec)


def _lane_total(v):
    idx = lax.iota(jnp.int32, 16)
    for sh in (8, 4, 2, 1):
        v = v + _gather_idx(v, jnp.bitwise_xor(idx, sh))
    return v


def _gather_idx(v, idxvec):
    dn = lax.GatherDimensionNumbers(
        offset_dims=(), collapsed_slice_dims=(0,), start_index_map=(0,))
    return lax.gather(
        v, idxvec.reshape(16, 1), dn, slice_sizes=(1,),
        mode=lax.GatherScatterMode.PROMISE_IN_BOUNDS)


def _sc_mean_agg_body(x_hbm, px_hbm, py_hbm, out_hbm, px_v, py_v,
                      nbr_v, rows_v, out_v, sem):
    wid = lax.axis_index("s") * 2 + lax.axis_index("c")
    for t in range(_EPW):
        env = wid * _EPW + t
        pltpu.sync_copy(px_hbm.at[env], px_v)
        pltpu.sync_copy(py_hbm.at[env], py_v)
        base = env * _N_AGENTS

        def dst_body(j, carry):
            lane = jnp.full((16,), j % 16, jnp.int32)
            pxj = _bcast_lane(px_v[pl.ds((j // 16) * 16, 16)], lane)
            pyj = _bcast_lane(py_v[pl.ds((j // 16) * 16, 16)], lane)

            iota = lax.iota(jnp.int32, 16)
            deg = jnp.zeros((16,), jnp.int32)
            for c in range(8):
                dx = px_v[pl.ds(c * 16, 16)] - pxj
                dy = py_v[pl.ds(c * 16, 16)] - pyj
                m = (dx * dx + dy * dy) <= _T2
                nbr_v[pl.ds(c * 16, 16)] = jnp.where(
                    m, base + c * 16 + iota, _ZROW)
                mi = jnp.where(m, 1, 0).astype(jnp.int32)
                deg = deg + mi

            zeros16 = jnp.zeros((16,), jnp.float32)
            for c in range(_FC):
                out_v[pl.ds(j * _F + c * 16, 16)] = zeros16

            def wave(w, carry2):
                idxv = nbr_v[pl.ds(w * 16, 16)]
                pltpu.async_copy(x_hbm.at[idxv], rows_v, sem).wait()
                for c in range(_FC):
                    acc = rows_v[0, pl.ds(c * 16, 16)]
                    for k in range(1, 16):
                        acc = acc + rows_v[k, pl.ds(c * 16, 16)]
                    plsc.addupdate(out_v.at[pl.ds(j * _F + c * 16, 16)],
                                   acc)
                return carry2

            lax.fori_loop(0, 8, wave, 0)

            rinv = 1.0 / _lane_total(deg).astype(jnp.float32)
            for c in range(_FC):
                off = j * _F + c * 16
                out_v[pl.ds(off, 16)] = out_v[pl.ds(off, 16)] * rinv
            return carry

        lax.fori_loop(0, _N_AGENTS, dst_body, 0)
        pltpu.sync_copy(
            out_v, out_hbm.at[pl.ds(base * _F, _N_AGENTS * _F)])


def _sc_mean_agg(xpad, px, py):
    n = _NUM_ENVS * _N_AGENTS
    mesh = plsc.VectorSubcoreMesh(core_axis_name="c", subcore_axis_name="s")
    k = pl.kernel(
        _sc_mean_agg_body,
        out_type=jax.ShapeDtypeStruct((n * _F,), jnp.float32),
        mesh=mesh,
        scratch_types=[
            pltpu.VMEM((_N_AGENTS,), jnp.float32),
            pltpu.VMEM((_N_AGENTS,), jnp.float32),
            pltpu.VMEM((144,), jnp.int32),
            pltpu.VMEM((16, _F), jnp.float32),
            pltpu.VMEM((_N_AGENTS * _F,), jnp.float32),
            pltpu.SemaphoreType.DMA,
        ],
    )
    return k(xpad, px, py).reshape(n, _F)


def _linear_relu_kernel(a_ref, w_ref, b_ref, o_ref):
    o_ref[...] = jnp.maximum(
        jnp.dot(a_ref[...], w_ref[...], preferred_element_type=jnp.float32)
        + b_ref[...], 0.0)


def _tc_linear_relu(a, w, b):
    n = a.shape[0]
    blk = 1024
    return pl.pallas_call(
        _linear_relu_kernel,
        grid=(n // blk,),
        in_specs=[
            pl.BlockSpec((blk, _F), lambda i: (i, 0)),
            pl.BlockSpec((_F, _HIDDEN_DIM), lambda i: (0, 0)),
            pl.BlockSpec((1, _HIDDEN_DIM), lambda i: (0, 0)),
        ],
        out_specs=pl.BlockSpec((blk, _HIDDEN_DIM), lambda i: (i, 0)),
        out_shape=jax.ShapeDtypeStruct((n, _HIDDEN_DIM), jnp.float32),
        compiler_params=pltpu.CompilerParams(
            dimension_semantics=("parallel",)),
    )(a, w, b.reshape(1, _HIDDEN_DIM))


def _pad_rows(a):
    return jnp.concatenate(
        [a, jnp.zeros((8, a.shape[1]), a.dtype)], axis=0)


def kernel(x, positions, W1, b1, W2, b2):
    n = _NUM_ENVS * _N_AGENTS
    x_flat = x.reshape(n, _F)
    px = positions[:, :, 0]
    py = positions[:, :, 1]

    agg1 = _sc_mean_agg(_pad_rows(x_flat), px, py)
    h1 = _tc_linear_relu(agg1, W1, b1)
    agg2 = _sc_mean_agg(_pad_rows(h1), px, py)
    h2 = _tc_linear_relu(agg2, W2, b2)
    return h2


# speedup vs baseline: 2977.5149x; 2977.5149x over previous
import jax
import jax.numpy as jnp
from jax.experimental import pallas as pl
from jax.experimental.pallas import tpu as pltpu

_OBS_DIM = 512
_HIDDEN_DIM = 512
_NUM_ENVS = 64
_N_AGENTS = 128
_DIST = 0.2
_T2 = 0.04000000283122063
_EPP = 16


def _fused_env_kernel(pos_ref, post_ref, x_ref, w1_ref, b1_ref, w2_ref,
                      b2_ref, out_ref):
    masks = []
    degs = []
    aggs = []
    for i in range(_EPP):
        pos = pos_ref[i]
        post = post_ref[i]
        dx = pos[:, 0:1] - post[0:1, :]
        dy = pos[:, 1:2] - post[1:2, :]
        maskf = (dx * dx + dy * dy <= _T2).astype(jnp.float32)
        deg = jnp.maximum(jnp.sum(maskf, axis=1, keepdims=True), 1.0)
        masks.append(maskf)
        degs.append(deg)
        aggs.append(
            jnp.dot(maskf, x_ref[i], preferred_element_type=jnp.float32)
            / deg)

    agg = jnp.concatenate(aggs, axis=0)
    h1 = jnp.maximum(
        jnp.dot(agg, w1_ref[...], preferred_element_type=jnp.float32)
        + b1_ref[...], 0.0)

    aggs2 = []
    for i in range(_EPP):
        h1i = h1[i * _N_AGENTS:(i + 1) * _N_AGENTS, :]
        aggs2.append(
            jnp.dot(masks[i], h1i, preferred_element_type=jnp.float32)
            / degs[i])
    agg2 = jnp.concatenate(aggs2, axis=0)
    h2 = jnp.maximum(
        jnp.dot(agg2, w2_ref[...], preferred_element_type=jnp.float32)
        + b2_ref[...], 0.0)

    out_ref[...] = h2


def kernel(x, positions, W1, b1, W2, b2):
    num_envs, n_agents, feat = x.shape
    pos_t = positions.transpose(0, 2, 1)
    b1r = b1.reshape(1, _HIDDEN_DIM)
    b2r = b2.reshape(1, _HIDDEN_DIM)

    out = pl.pallas_call(
        _fused_env_kernel,
        grid=(num_envs // _EPP,),
        in_specs=[
            pl.BlockSpec((_EPP, n_agents, 2), lambda e: (e, 0, 0)),
            pl.BlockSpec((_EPP, 2, n_agents), lambda e: (e, 0, 0)),
            pl.BlockSpec((_EPP, n_agents, feat), lambda e: (e, 0, 0)),
            pl.BlockSpec((feat, _HIDDEN_DIM), lambda e: (0, 0)),
            pl.BlockSpec((1, _HIDDEN_DIM), lambda e: (0, 0)),
            pl.BlockSpec((_HIDDEN_DIM, _HIDDEN_DIM), lambda e: (0, 0)),
            pl.BlockSpec((1, _HIDDEN_DIM), lambda e: (0, 0)),
        ],
        out_specs=pl.BlockSpec((_EPP * n_agents, _HIDDEN_DIM),
                               lambda e: (e, 0)),
        out_shape=jax.ShapeDtypeStruct((num_envs * n_agents, _HIDDEN_DIM),
                                       jnp.float32),
        compiler_params=pltpu.CompilerParams(
            dimension_semantics=("parallel",)),
    )(positions, pos_t, x, W1, b1r, W2, b2r)
    return out
